# baseline (speedup 1.0000x reference)
import functools

import jax
import jax.numpy as jnp
from jax import lax
from jax.experimental import pallas as pl
from jax.experimental.pallas import tpu as pltpu
from jax.experimental.pallas import tpu_sc as plsc

_B, _N, _C = 4, 8192, 128
_S, _NS = 512, 32
_R2 = 0.2 ** 2
_COUT = 512

_NW = 32
_QW = (_B * _S) // _NW
_OB = 96
_ROWS_W = (_B * _S * _NS) // _NW
_GCH = 128

_TM = 2048


def _fps_body(x_ref, idx_ref, xyz_ref):
    iota3 = (lax.broadcasted_iota(jnp.int32, (_B, 64, 128), 1) * 128
             + lax.broadcasted_iota(jnp.int32, (_B, 64, 128), 2))
    lane8 = lax.broadcasted_iota(jnp.int32, (1, 8), 1)
    lane16 = lax.broadcasted_iota(jnp.int32, (1, 16), 1)
    xs = [[x_ref[3 * b + c] for c in range(3)] for b in range(_B)]
    x3 = [jnp.stack([xs[b][c] for b in range(_B)]) for c in range(3)]

    def emit(i, nxts, qs):
        iv = jnp.zeros((1, 8), jnp.int32)
        for b in range(_B):
            iv = iv + jnp.where(lane8 == b, nxts[b], 0)
        xv = jnp.zeros((1, 16), jnp.float32)
        for b in range(_B):
            for c in range(3):
                xv = xv + jnp.where(lane16 == (c * 4 + b), qs[b][c], 0.0)
        idx_ref[pl.ds(i, 1), :] = iv
        xyz_ref[pl.ds(i, 1), :] = xv

    q0 = tuple(tuple(xs[b][c][0, 0] for c in range(3)) for b in range(_B))
    emit(0, (jnp.int32(0),) * _B, q0)

    def step(i, carry):
        dists, qs = carry
        dbs = []
        for b in range(_B):
            qx, qy, qz = qs[b]
            dx = xs[b][0] - qx
            dy = xs[b][1] - qy
            dz = xs[b][2] - qz
            d = dx * dx + dy * dy + dz * dz
            dbs.append(jnp.minimum(dists[b], d))
        d3 = jnp.stack(dbs)
        m3 = jnp.max(d3, axis=(1, 2), keepdims=True)
        n3 = jnp.min(jnp.where(d3 == m3, iota3, _N), axis=(1, 2), keepdims=True)
        sel1 = iota3 == n3
        qc3 = [jnp.sum(jnp.where(sel1, x3[c], 0.0), axis=(1, 2), keepdims=True)
               for c in range(3)]
        nxts = [n3[b, 0, 0] for b in range(_B)]
        nqs = tuple(tuple(qc3[c][b, 0, 0] for c in range(3)) for b in range(_B))
        emit(i, nxts, nqs)
        return tuple(dbs), nqs

    dists0 = tuple(jnp.full((64, 128), 1e10, jnp.float32) for _ in range(_B))
    lax.fori_loop(1, _S, step, (dists0, q0))


def _fps(xr):
    return pl.pallas_call(
        _fps_body,
        out_shape=[jax.ShapeDtypeStruct((_S, 8), jnp.int32),
                   jax.ShapeDtypeStruct((_S, 16), jnp.float32)],
    )(xr)


def _ballq_body(xyz_hbm, q_hbm, idx_hbm, idxg_hbm, gx_hbm,
                xvx, xvy, xvz, qv, obuf, idxv, idxgv, gxv, cnt_ref):
    wid = lax.axis_index("s") * 2 + lax.axis_index("c")
    b = wid // (_NW // _B)
    pltpu.sync_copy(xyz_hbm.at[pl.ds((b * 3 + 0) * _N, _N)], xvx)
    pltpu.sync_copy(xyz_hbm.at[pl.ds((b * 3 + 1) * _N, _N)], xvy)
    pltpu.sync_copy(xyz_hbm.at[pl.ds((b * 3 + 2) * _N, _N)], xvz)
    pltpu.sync_copy(q_hbm.at[pl.ds(wid * _QW, _QW)], qv)
    iota16 = lax.iota(jnp.int32, 16)
    nblk = _N // 64

    def per_query(i, carry):
        qrow = qv[i]
        qx = qrow[0]
        qy = qrow[1]
        qz = qrow[2]

        cnt_ref[0] = jnp.int32(0)

        def blk_body(blk, carry2):
            @pl.when(cnt_ref[0] < _NS)
            def _():
                cur = cnt_ref[0]
                for u in range(4):
                    base = blk * 64 + u * 16
                    px = xvx[pl.ds(base, 16)]
                    py = xvy[pl.ds(base, 16)]
                    pz = xvz[pl.ds(base, 16)]
                    dx = px - qx
                    dy = py - qy
                    dz = pz - qz
                    d2 = dx * dx + dy * dy + dz * dz
                    msk = d2 < _R2
                    plsc.store_compressed(obuf.at[pl.ds(cur, 16)],
                                          iota16 + base, mask=msk)
                    pc = plsc.all_reduce_population_count(msk)
                    cur = cur + pc[0]
                cnt_ref[0] = cur
            return carry2

        lax.fori_loop(0, nblk, blk_body, 0)
        cnt = cnt_ref[0]
        v0 = obuf[pl.ds(0, 16)]
        v1 = obuf[pl.ds(16, 16)]
        first = jnp.where(cnt > 0, v0[0], 0)
        o0 = jnp.where(iota16 < cnt, v0, first)
        o1 = jnp.where(iota16 + 16 < cnt, v1, first)
        idxv[pl.ds(i * _NS, 16)] = o0
        idxv[pl.ds(i * _NS + 16, 16)] = o1
        idxgv[pl.ds(i * _NS, 16)] = o0 + b * _N
        idxgv[pl.ds(i * _NS + 16, 16)] = o1 + b * _N
        for h, ov in ((0, o0), (1, o1)):
            gx = plsc.load_gather(xvx, [ov]) - qx
            gy = plsc.load_gather(xvy, [ov]) - qy
            gz = plsc.load_gather(xvz, [ov]) - qz
            pos3 = iota16 * 3 + (i * (3 * _NS) + h * 48)
            plsc.store_scatter(gxv, [pos3], gx)
            plsc.store_scatter(gxv, [pos3 + 1], gy)
            plsc.store_scatter(gxv, [pos3 + 2], gz)
        return carry

    lax.fori_loop(0, _QW, per_query, 0)
    pltpu.sync_copy(idxv, idx_hbm.at[pl.ds(wid * _QW * _NS, _QW * _NS)])
    pltpu.sync_copy(idxgv, idxg_hbm.at[pl.ds(wid * _QW * _NS, _QW * _NS)])
    pltpu.sync_copy(gxv, gx_hbm.at[pl.ds(wid * _QW * _NS * 3, _QW * _NS * 3)])


def _ballq(xyz_t, q16):
    f = pl.kernel(
        _ballq_body,
        out_type=[jax.ShapeDtypeStruct((_B * _S * _NS,), jnp.int32),
                  jax.ShapeDtypeStruct((_B * _S * _NS,), jnp.int32),
                  jax.ShapeDtypeStruct((_B * _S * _NS * 3,), jnp.float32)],
        mesh=plsc.VectorSubcoreMesh(core_axis_name="c", subcore_axis_name="s"),
        scratch_types=[
            pltpu.VMEM((_N,), jnp.float32),
            pltpu.VMEM((_N,), jnp.float32),
            pltpu.VMEM((_N,), jnp.float32),
            pltpu.VMEM((_QW, 16), jnp.float32),
            pltpu.VMEM((_OB,), jnp.int32),
            pltpu.VMEM((_QW * _NS,), jnp.int32),
            pltpu.VMEM((_QW * _NS,), jnp.int32),
            pltpu.VMEM((_QW * _NS * 3,), jnp.float32),
            pltpu.SMEM((1,), jnp.int32),
        ],
        compiler_params=pltpu.CompilerParams(needs_layout_passes=False),
    )
    return f(xyz_t, q16)


def _gather_body(ft_hbm, idxg_hbm, out_hbm, idx1, buf0, buf1, sem0, sem1):
    wid = lax.axis_index("s") * 2 + lax.axis_index("c")
    base = wid * _ROWS_W
    pltpu.sync_copy(idxg_hbm.at[pl.ds(base, _ROWS_W)], idx1)
    bufs = (buf0, buf1)
    sems = (sem0, sem1)
    nch = _ROWS_W // _GCH

    def start(ck):
        return pltpu.async_copy(
            ft_hbm.at[idx1.at[pl.ds(ck * _GCH, _GCH)]], bufs[ck % 2], sems[ck % 2])

    cp = start(0)
    for ck in range(nch):
        nxt = start(ck + 1) if ck + 1 < nch else None
        cp.wait()
        pltpu.sync_copy(bufs[ck % 2], out_hbm.at[pl.ds(base + ck * _GCH, _GCH)])
        cp = nxt


def _gather(ft, idxg):
    f = pl.kernel(
        _gather_body,
        out_type=[jax.ShapeDtypeStruct((_B * _S * _NS, _C), jnp.float32)],
        mesh=plsc.VectorSubcoreMesh(core_axis_name="c", subcore_axis_name="s"),
        scratch_types=[
            pltpu.VMEM((_ROWS_W,), jnp.int32),
            pltpu.VMEM((_GCH, _C), jnp.float32),
            pltpu.VMEM((_GCH, _C), jnp.float32),
            pltpu.SemaphoreType.DMA,
            pltpu.SemaphoreType.DMA,
        ],
        compiler_params=pltpu.CompilerParams(needs_layout_passes=False),
    )
    return f(ft, idxg)[0]


def _mlp_body(f_ref, g_ref, w1f_ref, w1x_ref, b1_ref, w2_ref, b2_ref,
              w3_ref, b3_ref, o_ref):
    f = f_ref[...]
    g = g_ref[...]
    h = jnp.dot(f, w1f_ref[...], preferred_element_type=jnp.float32)
    h = h + jnp.dot(g, w1x_ref[...], preferred_element_type=jnp.float32)
    h = jnp.maximum(h + b1_ref[...], 0.0)
    h = jnp.maximum(jnp.dot(h, w2_ref[...], preferred_element_type=jnp.float32)
                    + b2_ref[...], 0.0)
    h = jnp.maximum(jnp.dot(h, w3_ref[...], preferred_element_type=jnp.float32)
                    + b3_ref[...], 0.0)
    o_ref[...] = jnp.max(h.reshape(_TM // _NS, _NS, _COUT), axis=1)


def _mlp(gfeat, gx, w1f, w1x, b1, w2, b2, w3, b3):
    nrows = _B * _S * _NS
    grid = (nrows // _TM,)
    return pl.pallas_call(
        _mlp_body,
        grid=grid,
        in_specs=[
            pl.BlockSpec((_TM, _C), lambda i: (i, 0)),
            pl.BlockSpec((_TM, 3), lambda i: (i, 0)),
            pl.BlockSpec((_C, 128), lambda i: (0, 0)),
            pl.BlockSpec((3, 128), lambda i: (0, 0)),
            pl.BlockSpec((1, 128), lambda i: (0, 0)),
            pl.BlockSpec((128, 256), lambda i: (0, 0)),
            pl.BlockSpec((1, 256), lambda i: (0, 0)),
            pl.BlockSpec((256, _COUT), lambda i: (0, 0)),
            pl.BlockSpec((1, _COUT), lambda i: (0, 0)),
        ],
        out_specs=pl.BlockSpec((_TM // _NS, _COUT), lambda i: (i, 0)),
        out_shape=jax.ShapeDtypeStruct((nrows // _NS, _COUT), jnp.float32),
    )(gfeat, gx, w1f, w1x, b1, w2, b2, w3, b3)


def kernel(xyz, features, W1, b1, W2, b2, W3, b3):
    xr = xyz.transpose(0, 2, 1).reshape(_B * 3, 64, 128)
    idx8, xyz16 = _fps(xr)
    idx_fps = idx8[:, :_B].T.astype(jnp.int64)
    new_xyz = xyz16[:, :12].reshape(_S, 3, _B).transpose(2, 0, 1)
    q16 = jnp.pad(new_xyz.reshape(_B * _S, 3), ((0, 0), (0, 13)))
    wpb = _NW // _B
    jpb = _S // wpb
    q16 = q16.reshape(_B, jpb, wpb, 16).transpose(0, 2, 1, 3).reshape(_B * _S, 16)
    xyz_t = xyz.transpose(0, 2, 1).reshape(_B * 3 * _N)
    idxf, idxg, gxf = _ballq(xyz_t, q16)

    def _unshuf(a, tail):
        return (a.reshape(_B, wpb, jpb, tail).transpose(0, 2, 1, 3)
                .reshape(_B * _S, tail))

    idx = _unshuf(idxf, _NS).reshape(_B, _S, _NS)
    idxg = _unshuf(idxg, _NS).reshape(_B * _S * _NS)
    ft = features.transpose(0, 2, 1).reshape(_B * _N, _C)
    gfeat = _gather(ft, idxg)
    gx = _unshuf(gxf, _NS * 3).reshape(_B * _S * _NS, 3)
    pooled = _mlp(gfeat, gx, W1[3:], W1[:3], b1.reshape(1, -1),
                  W2, b2.reshape(1, -1), W3, b3.reshape(1, -1))
    new_features = pooled.reshape(_B, _S, _COUT).transpose(0, 2, 1)
    return (new_xyz, idx_fps, new_features, idx)


# speedup vs baseline: 1.1111x; 1.1111x over previous
import functools

import jax
import jax.numpy as jnp
from jax import lax
from jax.experimental import pallas as pl
from jax.experimental.pallas import tpu as pltpu
from jax.experimental.pallas import tpu_sc as plsc

_B, _N, _C = 4, 8192, 128
_S, _NS = 512, 32
_R2 = 0.2 ** 2
_COUT = 512

_NW = 32
_QW = (_B * _S) // _NW
_OB = 96
_ROWS_W = (_B * _S * _NS) // _NW
_GCH = 128

_TM = 2048


def _fps_body(x_ref, idx_ref, xyz_ref):
    iota3 = (lax.broadcasted_iota(jnp.int32, (_B, 64, 128), 1) * 128
             + lax.broadcasted_iota(jnp.int32, (_B, 64, 128), 2))
    lane8 = lax.broadcasted_iota(jnp.int32, (1, 8), 1)
    lane16 = lax.broadcasted_iota(jnp.int32, (1, 16), 1)
    xs = [[x_ref[3 * b + c] for c in range(3)] for b in range(_B)]
    x3 = [jnp.stack([xs[b][c] for b in range(_B)]) for c in range(3)]

    def emit(i, nxts, qs):
        iv = jnp.zeros((1, 8), jnp.int32)
        for b in range(_B):
            iv = iv + jnp.where(lane8 == b, nxts[b], 0)
        xv = jnp.zeros((1, 16), jnp.float32)
        for b in range(_B):
            for c in range(3):
                xv = xv + jnp.where(lane16 == (c * 4 + b), qs[b][c], 0.0)
        idx_ref[pl.ds(i, 1), :] = iv
        xyz_ref[pl.ds(i, 1), :] = xv

    q0 = tuple(tuple(xs[b][c][0, 0] for c in range(3)) for b in range(_B))
    emit(0, (jnp.int32(0),) * _B, q0)

    def step(i, carry):
        dists, qs = carry
        dbs = []
        for b in range(_B):
            qx, qy, qz = qs[b]
            dx = xs[b][0] - qx
            dy = xs[b][1] - qy
            dz = xs[b][2] - qz
            d = dx * dx + dy * dy + dz * dz
            dbs.append(jnp.minimum(dists[b], d))
        d3 = jnp.stack(dbs)
        m3 = jnp.max(d3, axis=(1, 2), keepdims=True)
        n3 = jnp.min(jnp.where(d3 == m3, iota3, _N), axis=(1, 2), keepdims=True)
        sel1 = iota3 == n3
        qc3 = [jnp.sum(jnp.where(sel1, x3[c], 0.0), axis=(1, 2), keepdims=True)
               for c in range(3)]
        nxts = [n3[b, 0, 0] for b in range(_B)]
        nqs = tuple(tuple(qc3[c][b, 0, 0] for c in range(3)) for b in range(_B))
        emit(i, nxts, nqs)
        return tuple(dbs), nqs

    dists0 = tuple(jnp.full((64, 128), 1e10, jnp.float32) for _ in range(_B))
    lax.fori_loop(1, _S, step, (dists0, q0))


def _fps(xr):
    return pl.pallas_call(
        _fps_body,
        out_shape=[jax.ShapeDtypeStruct((_S, 8), jnp.int32),
                   jax.ShapeDtypeStruct((_S, 16), jnp.float32)],
    )(xr)


def _ballq_body(xyz_hbm, q_hbm, idx_hbm, idxg_hbm, gx_hbm,
                xvx, xvy, xvz, qv, obuf, idxv, idxgv, gxv, cnt_ref):
    wid = lax.axis_index("s") * 2 + lax.axis_index("c")
    b = wid // (_NW // _B)
    pltpu.sync_copy(xyz_hbm.at[pl.ds((b * 3 + 0) * _N, _N)], xvx)
    pltpu.sync_copy(xyz_hbm.at[pl.ds((b * 3 + 1) * _N, _N)], xvy)
    pltpu.sync_copy(xyz_hbm.at[pl.ds((b * 3 + 2) * _N, _N)], xvz)
    pltpu.sync_copy(q_hbm.at[pl.ds(wid * _QW, _QW)], qv)
    iota16 = lax.iota(jnp.int32, 16)
    nblk = _N // 64

    def per_query(i, carry):
        qrow = qv[i]
        qx = qrow[0]
        qy = qrow[1]
        qz = qrow[2]

        cnt_ref[0] = jnp.int32(0)

        def blk_body(blk, carry2):
            @pl.when(cnt_ref[0] < _NS)
            def _():
                msks, pcs = [], []
                for u in range(4):
                    base = blk * 64 + u * 16
                    px = xvx[pl.ds(base, 16)]
                    py = xvy[pl.ds(base, 16)]
                    pz = xvz[pl.ds(base, 16)]
                    dx = px - qx
                    dy = py - qy
                    dz = pz - qz
                    d2 = dx * dx + dy * dy + dz * dz
                    msk = d2 < _R2
                    msks.append(msk)
                    pcs.append(plsc.all_reduce_population_count(msk)[0])
                cur = cnt_ref[0]
                for u in range(4):
                    plsc.store_compressed(obuf.at[pl.ds(cur, 16)],
                                          iota16 + blk * 64 + u * 16, mask=msks[u])
                    cur = cur + pcs[u]
                cnt_ref[0] = cur
            return carry2

        lax.fori_loop(0, nblk, blk_body, 0)
        cnt = cnt_ref[0]
        v0 = obuf[pl.ds(0, 16)]
        v1 = obuf[pl.ds(16, 16)]
        first = jnp.where(cnt > 0, v0[0], 0)
        o0 = jnp.where(iota16 < cnt, v0, first)
        o1 = jnp.where(iota16 + 16 < cnt, v1, first)
        idxv[pl.ds(i * _NS, 16)] = o0
        idxv[pl.ds(i * _NS + 16, 16)] = o1
        idxgv[pl.ds(i * _NS, 16)] = o0 + b * _N
        idxgv[pl.ds(i * _NS + 16, 16)] = o1 + b * _N
        for h, ov in ((0, o0), (1, o1)):
            gx = plsc.load_gather(xvx, [ov]) - qx
            gy = plsc.load_gather(xvy, [ov]) - qy
            gz = plsc.load_gather(xvz, [ov]) - qz
            pos3 = iota16 * 3 + (i * (3 * _NS) + h * 48)
            plsc.store_scatter(gxv, [pos3], gx)
            plsc.store_scatter(gxv, [pos3 + 1], gy)
            plsc.store_scatter(gxv, [pos3 + 2], gz)
        return carry

    lax.fori_loop(0, _QW, per_query, 0)
    pltpu.sync_copy(idxv, idx_hbm.at[pl.ds(wid * _QW * _NS, _QW * _NS)])
    pltpu.sync_copy(idxgv, idxg_hbm.at[pl.ds(wid * _QW * _NS, _QW * _NS)])
    pltpu.sync_copy(gxv, gx_hbm.at[pl.ds(wid * _QW * _NS * 3, _QW * _NS * 3)])


def _ballq(xyz_t, q16):
    f = pl.kernel(
        _ballq_body,
        out_type=[jax.ShapeDtypeStruct((_B * _S * _NS,), jnp.int32),
                  jax.ShapeDtypeStruct((_B * _S * _NS,), jnp.int32),
                  jax.ShapeDtypeStruct((_B * _S * _NS * 3,), jnp.float32)],
        mesh=plsc.VectorSubcoreMesh(core_axis_name="c", subcore_axis_name="s"),
        scratch_types=[
            pltpu.VMEM((_N,), jnp.float32),
            pltpu.VMEM((_N,), jnp.float32),
            pltpu.VMEM((_N,), jnp.float32),
            pltpu.VMEM((_QW, 16), jnp.float32),
            pltpu.VMEM((_OB,), jnp.int32),
            pltpu.VMEM((_QW * _NS,), jnp.int32),
            pltpu.VMEM((_QW * _NS,), jnp.int32),
            pltpu.VMEM((_QW * _NS * 3,), jnp.float32),
            pltpu.SMEM((1,), jnp.int32),
        ],
        compiler_params=pltpu.CompilerParams(needs_layout_passes=False),
    )
    return f(xyz_t, q16)


def _gather_body(ft_hbm, idxg_hbm, out_hbm, idx1, buf0, buf1, sem0, sem1):
    wid = lax.axis_index("s") * 2 + lax.axis_index("c")
    base = wid * _ROWS_W
    pltpu.sync_copy(idxg_hbm.at[pl.ds(base, _ROWS_W)], idx1)
    bufs = (buf0, buf1)
    sems = (sem0, sem1)
    nch = _ROWS_W // _GCH

    def start(ck):
        return pltpu.async_copy(
            ft_hbm.at[idx1.at[pl.ds(ck * _GCH, _GCH)]], bufs[ck % 2], sems[ck % 2])

    cp = start(0)
    for ck in range(nch):
        nxt = start(ck + 1) if ck + 1 < nch else None
        cp.wait()
        pltpu.sync_copy(bufs[ck % 2], out_hbm.at[pl.ds(base + ck * _GCH, _GCH)])
        cp = nxt


def _gather(ft, idxg):
    f = pl.kernel(
        _gather_body,
        out_type=[jax.ShapeDtypeStruct((_B * _S * _NS, _C), jnp.float32)],
        mesh=plsc.VectorSubcoreMesh(core_axis_name="c", subcore_axis_name="s"),
        scratch_types=[
            pltpu.VMEM((_ROWS_W,), jnp.int32),
            pltpu.VMEM((_GCH, _C), jnp.float32),
            pltpu.VMEM((_GCH, _C), jnp.float32),
            pltpu.SemaphoreType.DMA,
            pltpu.SemaphoreType.DMA,
        ],
        compiler_params=pltpu.CompilerParams(needs_layout_passes=False),
    )
    return f(ft, idxg)[0]


def _mlp_body(f_ref, g_ref, w1f_ref, w1x_ref, b1_ref, w2_ref, b2_ref,
              w3_ref, b3_ref, o_ref):
    f = f_ref[...]
    g = g_ref[...]
    h = jnp.dot(f, w1f_ref[...], preferred_element_type=jnp.float32)
    h = h + jnp.dot(g, w1x_ref[...], preferred_element_type=jnp.float32)
    h = jnp.maximum(h + b1_ref[...], 0.0)
    h = jnp.maximum(jnp.dot(h, w2_ref[...], preferred_element_type=jnp.float32)
                    + b2_ref[...], 0.0)
    h = jnp.maximum(jnp.dot(h, w3_ref[...], preferred_element_type=jnp.float32)
                    + b3_ref[...], 0.0)
    o_ref[...] = jnp.max(h.reshape(_TM // _NS, _NS, _COUT), axis=1)


def _mlp(gfeat, gx, w1f, w1x, b1, w2, b2, w3, b3):
    nrows = _B * _S * _NS
    grid = (nrows // _TM,)
    return pl.pallas_call(
        _mlp_body,
        grid=grid,
        in_specs=[
            pl.BlockSpec((_TM, _C), lambda i: (i, 0)),
            pl.BlockSpec((_TM, 3), lambda i: (i, 0)),
            pl.BlockSpec((_C, 128), lambda i: (0, 0)),
            pl.BlockSpec((3, 128), lambda i: (0, 0)),
            pl.BlockSpec((1, 128), lambda i: (0, 0)),
            pl.BlockSpec((128, 256), lambda i: (0, 0)),
            pl.BlockSpec((1, 256), lambda i: (0, 0)),
            pl.BlockSpec((256, _COUT), lambda i: (0, 0)),
            pl.BlockSpec((1, _COUT), lambda i: (0, 0)),
        ],
        out_specs=pl.BlockSpec((_TM // _NS, _COUT), lambda i: (i, 0)),
        out_shape=jax.ShapeDtypeStruct((nrows // _NS, _COUT), jnp.float32),
    )(gfeat, gx, w1f, w1x, b1, w2, b2, w3, b3)


def kernel(xyz, features, W1, b1, W2, b2, W3, b3):
    xr = xyz.transpose(0, 2, 1).reshape(_B * 3, 64, 128)
    idx8, xyz16 = _fps(xr)
    idx_fps = idx8[:, :_B].T.astype(jnp.int64)
    new_xyz = xyz16[:, :12].reshape(_S, 3, _B).transpose(2, 0, 1)
    q16 = jnp.pad(new_xyz.reshape(_B * _S, 3), ((0, 0), (0, 13)))
    wpb = _NW // _B
    jpb = _S // wpb
    q16 = q16.reshape(_B, jpb, wpb, 16).transpose(0, 2, 1, 3).reshape(_B * _S, 16)
    xyz_t = xyz.transpose(0, 2, 1).reshape(_B * 3 * _N)
    idxf, idxg, gxf = _ballq(xyz_t, q16)

    def _unshuf(a, tail):
        return (a.reshape(_B, wpb, jpb, tail).transpose(0, 2, 1, 3)
                .reshape(_B * _S, tail))

    idx = _unshuf(idxf, _NS).reshape(_B, _S, _NS)
    idxg = _unshuf(idxg, _NS).reshape(_B * _S * _NS)
    ft = features.transpose(0, 2, 1).reshape(_B * _N, _C)
    gfeat = _gather(ft, idxg)
    gx = _unshuf(gxf, _NS * 3).reshape(_B * _S * _NS, 3)
    pooled = _mlp(gfeat, gx, W1[3:], W1[:3], b1.reshape(1, -1),
                  W2, b2.reshape(1, -1), W3, b3.reshape(1, -1))
    new_features = pooled.reshape(_B, _S, _COUT).transpose(0, 2, 1)
    return (new_xyz, idx_fps, new_features, idx)


# speedup vs baseline: 1.1181x; 1.0063x over previous
import jax
import jax.numpy as jnp
from jax import lax
from jax.experimental import pallas as pl
from jax.experimental.pallas import tpu as pltpu
from jax.experimental.pallas import tpu_sc as plsc

_B, _N, _C = 4, 8192, 128
_S, _NS = 512, 32
_R2 = 0.2 ** 2
_COUT = 512

_SH = _S // 2
_NW = 32
_WPB = _NW // _B
_QW = _SH // _WPB
_OB = 96
_GCH = 128

_TM = 2048


def _fps_prelude(x_ref, idx_ref, xyz_ref):
    iota3 = (lax.broadcasted_iota(jnp.int32, (_B, 64, 128), 1) * 128
             + lax.broadcasted_iota(jnp.int32, (_B, 64, 128), 2))
    lane8 = lax.broadcasted_iota(jnp.int32, (1, 8), 1)
    lane16 = lax.broadcasted_iota(jnp.int32, (1, 16), 1)
    xs = [[x_ref[3 * b + c] for c in range(3)] for b in range(_B)]
    x3 = [jnp.stack([xs[b][c] for b in range(_B)]) for c in range(3)]

    def qvec(qs):
        xv = jnp.zeros((1, 16), jnp.float32)
        for b in range(_B):
            for c in range(3):
                xv = xv + jnp.where(lane16 == (c * 4 + b), qs[b][c], 0.0)
        return xv

    def emit(i, nxts, qs):
        iv = jnp.zeros((1, 8), jnp.int32)
        for b in range(_B):
            iv = iv + jnp.where(lane8 == b, nxts[b], 0)
        idx_ref[pl.ds(i, 1), :] = iv
        xyz_ref[pl.ds(i, 1), :] = qvec(qs)

    def step(i, carry):
        dists, qs = carry
        dbs = []
        for b in range(_B):
            qx, qy, qz = qs[b]
            dx = xs[b][0] - qx
            dy = xs[b][1] - qy
            dz = xs[b][2] - qz
            d = dx * dx + dy * dy + dz * dz
            dbs.append(jnp.minimum(dists[b], d))
        d3 = jnp.stack(dbs)
        m3 = jnp.max(d3, axis=(1, 2), keepdims=True)
        n3 = jnp.min(jnp.where(d3 == m3, iota3, _N), axis=(1, 2), keepdims=True)
        sel1 = iota3 == n3
        qc3 = [jnp.sum(jnp.where(sel1, x3[c], 0.0), axis=(1, 2), keepdims=True)
               for c in range(3)]
        nxts = [n3[b, 0, 0] for b in range(_B)]
        nqs = tuple(tuple(qc3[c][b, 0, 0] for c in range(3)) for b in range(_B))
        emit(i, nxts, nqs)
        return tuple(dbs), nqs

    return xs, qvec, emit, step


def _fps_a_body(x_ref, idx_ref, xyz_ref, ds_ref, qs_ref):
    xs, qvec, emit, step = _fps_prelude(x_ref, idx_ref, xyz_ref)
    q0 = tuple(tuple(xs[b][c][0, 0] for c in range(3)) for b in range(_B))
    emit(0, (jnp.int32(0),) * _B, q0)
    dists0 = tuple(jnp.full((64, 128), 1e10, jnp.float32) for _ in range(_B))
    dists, qs = lax.fori_loop(1, _SH, step, (dists0, q0))
    for b in range(_B):
        ds_ref[pl.ds(b * 64, 64), :] = dists[b]
    qs_ref[pl.ds(0, 1), :] = qvec(qs)


def _fps_a(xr):
    return pl.pallas_call(
        _fps_a_body,
        out_shape=[jax.ShapeDtypeStruct((_SH, 8), jnp.int32),
                   jax.ShapeDtypeStruct((_SH, 16), jnp.float32),
                   jax.ShapeDtypeStruct((_B * 64, 128), jnp.float32),
                   jax.ShapeDtypeStruct((1, 16), jnp.float32)],
    )(xr)


def _fps_b_body(x_ref, ds_ref, qs_ref, idx_ref, xyz_ref):
    xs, qvec, emit, step = _fps_prelude(x_ref, idx_ref, xyz_ref)
    dists0 = tuple(ds_ref[pl.ds(b * 64, 64), :] for b in range(_B))
    q0 = tuple(tuple(qs_ref[0, c * 4 + b] for c in range(3)) for b in range(_B))
    lax.fori_loop(0, _S - _SH, step, (dists0, q0))


def _fps_b(xr, ds, qs):
    return pl.pallas_call(
        _fps_b_body,
        out_shape=[jax.ShapeDtypeStruct((_S - _SH, 8), jnp.int32),
                   jax.ShapeDtypeStruct((_S - _SH, 16), jnp.float32)],
    )(xr, ds, qs)


def _ballq_body(xyz_hbm, q_hbm, idx_hbm, idxg_hbm, gx_hbm,
                xvx, xvy, xvz, qv, obuf, idxv, idxgv, gxv, cnt_ref):
    wid = lax.axis_index("s") * 2 + lax.axis_index("c")
    b = wid // _WPB
    pltpu.sync_copy(xyz_hbm.at[pl.ds((b * 3 + 0) * _N, _N)], xvx)
    pltpu.sync_copy(xyz_hbm.at[pl.ds((b * 3 + 1) * _N, _N)], xvy)
    pltpu.sync_copy(xyz_hbm.at[pl.ds((b * 3 + 2) * _N, _N)], xvz)
    pltpu.sync_copy(q_hbm.at[pl.ds(wid * _QW, _QW)], qv)
    iota16 = lax.iota(jnp.int32, 16)
    nblk = _N // 64

    def per_query(i, carry):
        qrow = qv[i]
        qx = qrow[0]
        qy = qrow[1]
        qz = qrow[2]
        cnt_ref[0] = jnp.int32(0)

        def blk_body(blk, carry2):
            @pl.when(cnt_ref[0] < _NS)
            def _():
                msks, pcs = [], []
                for u in range(4):
                    base = blk * 64 + u * 16
                    px = xvx[pl.ds(base, 16)]
                    py = xvy[pl.ds(base, 16)]
                    pz = xvz[pl.ds(base, 16)]
                    dx = px - qx
                    dy = py - qy
                    dz = pz - qz
                    d2 = dx * dx + dy * dy + dz * dz
                    msk = d2 < _R2
                    msks.append(msk)
                    pcs.append(plsc.all_reduce_population_count(msk)[0])
                cur = cnt_ref[0]
                for u in range(4):
                    plsc.store_compressed(obuf.at[pl.ds(cur, 16)],
                                          iota16 + blk * 64 + u * 16,
                                          mask=msks[u])
                    cur = cur + pcs[u]
                cnt_ref[0] = cur
            return carry2

        lax.fori_loop(0, nblk, blk_body, 0)
        cnt = cnt_ref[0]
        v0 = obuf[pl.ds(0, 16)]
        v1 = obuf[pl.ds(16, 16)]
        first = jnp.where(cnt > 0, v0[0], 0)
        o0 = jnp.where(iota16 < cnt, v0, first)
        o1 = jnp.where(iota16 + 16 < cnt, v1, first)
        idxv[pl.ds(i * _NS, 16)] = o0
        idxv[pl.ds(i * _NS + 16, 16)] = o1
        idxgv[pl.ds(i * _NS, 16)] = o0 + b * _N
        idxgv[pl.ds(i * _NS + 16, 16)] = o1 + b * _N
        for h, ov in ((0, o0), (1, o1)):
            gx = plsc.load_gather(xvx, [ov]) - qx
            gy = plsc.load_gather(xvy, [ov]) - qy
            gz = plsc.load_gather(xvz, [ov]) - qz
            pos3 = iota16 * 3 + (i * (3 * _NS) + h * 48)
            plsc.store_scatter(gxv, [pos3], gx)
            plsc.store_scatter(gxv, [pos3 + 1], gy)
            plsc.store_scatter(gxv, [pos3 + 2], gz)
        return carry

    lax.fori_loop(0, _QW, per_query, 0)
    pltpu.sync_copy(idxv, idx_hbm.at[pl.ds(wid * _QW * _NS, _QW * _NS)])
    pltpu.sync_copy(idxgv, idxg_hbm.at[pl.ds(wid * _QW * _NS, _QW * _NS)])
    pltpu.sync_copy(gxv, gx_hbm.at[pl.ds(wid * _QW * _NS * 3, _QW * _NS * 3)])


def _ballq(xyz_t, q16):
    n = _B * _SH * _NS
    f = pl.kernel(
        _ballq_body,
        out_type=[jax.ShapeDtypeStruct((n,), jnp.int32),
                  jax.ShapeDtypeStruct((n,), jnp.int32),
                  jax.ShapeDtypeStruct((n * 3,), jnp.float32)],
        mesh=plsc.VectorSubcoreMesh(core_axis_name="c", subcore_axis_name="s"),
        scratch_types=[
            pltpu.VMEM((_N,), jnp.float32),
            pltpu.VMEM((_N,), jnp.float32),
            pltpu.VMEM((_N,), jnp.float32),
            pltpu.VMEM((_QW, 16), jnp.float32),
            pltpu.VMEM((_OB,), jnp.int32),
            pltpu.VMEM((_QW * _NS,), jnp.int32),
            pltpu.VMEM((_QW * _NS,), jnp.int32),
            pltpu.VMEM((_QW * _NS * 3,), jnp.float32),
            pltpu.SMEM((1,), jnp.int32),
        ],
        compiler_params=pltpu.CompilerParams(needs_layout_passes=False),
    )
    return f(xyz_t, q16)


def _gather_body(ft_hbm, idxg_hbm, out_hbm, idx1, buf0, buf1, sem0, sem1):
    rows_w = (_B * _SH * _NS) // _NW
    wid = lax.axis_index("s") * 2 + lax.axis_index("c")
    base = wid * rows_w
    pltpu.sync_copy(idxg_hbm.at[pl.ds(base, rows_w)], idx1)
    bufs = (buf0, buf1)
    sems = (sem0, sem1)
    nch = rows_w // _GCH

    def start(ck):
        return pltpu.async_copy(
            ft_hbm.at[idx1.at[pl.ds(ck * _GCH, _GCH)]], bufs[ck % 2], sems[ck % 2])

    cp = start(0)
    for ck in range(nch):
        nxt = start(ck + 1) if ck + 1 < nch else None
        cp.wait()
        pltpu.sync_copy(bufs[ck % 2], out_hbm.at[pl.ds(base + ck * _GCH, _GCH)])
        cp = nxt


def _gather(ft, idxg):
    rows_w = (_B * _SH * _NS) // _NW
    f = pl.kernel(
        _gather_body,
        out_type=[jax.ShapeDtypeStruct((_B * _SH * _NS, _C), jnp.float32)],
        mesh=plsc.VectorSubcoreMesh(core_axis_name="c", subcore_axis_name="s"),
        scratch_types=[
            pltpu.VMEM((rows_w,), jnp.int32),
            pltpu.VMEM((_GCH, _C), jnp.float32),
            pltpu.VMEM((_GCH, _C), jnp.float32),
            pltpu.SemaphoreType.DMA,
            pltpu.SemaphoreType.DMA,
        ],
        compiler_params=pltpu.CompilerParams(needs_layout_passes=False),
    )
    return f(ft, idxg)[0]


def _mlp_body(f_ref, g_ref, w1f_ref, w1x_ref, b1_ref, w2_ref, b2_ref,
              w3_ref, b3_ref, o_ref):
    f = f_ref[...]
    g = g_ref[...]
    h = jnp.dot(f, w1f_ref[...], preferred_element_type=jnp.float32)
    h = h + jnp.dot(g, w1x_ref[...], preferred_element_type=jnp.float32)
    h = jnp.maximum(h + b1_ref[...], 0.0)
    h = jnp.maximum(jnp.dot(h, w2_ref[...], preferred_element_type=jnp.float32)
                    + b2_ref[...], 0.0)
    h = jnp.maximum(jnp.dot(h, w3_ref[...], preferred_element_type=jnp.float32)
                    + b3_ref[...], 0.0)
    o_ref[...] = jnp.max(h.reshape(_TM // _NS, _NS, _COUT), axis=1)


def _mlp(gfeat, gx, w1f, w1x, b1, w2, b2, w3, b3):
    nrows = _B * _SH * _NS
    grid = (nrows // _TM,)
    return pl.pallas_call(
        _mlp_body,
        grid=grid,
        in_specs=[
            pl.BlockSpec((_TM, _C), lambda i: (i, 0)),
            pl.BlockSpec((_TM, 3), lambda i: (i, 0)),
            pl.BlockSpec((_C, 128), lambda i: (0, 0)),
            pl.BlockSpec((3, 128), lambda i: (0, 0)),
            pl.BlockSpec((1, 128), lambda i: (0, 0)),
            pl.BlockSpec((128, 256), lambda i: (0, 0)),
            pl.BlockSpec((1, 256), lambda i: (0, 0)),
            pl.BlockSpec((256, _COUT), lambda i: (0, 0)),
            pl.BlockSpec((1, _COUT), lambda i: (0, 0)),
        ],
        out_specs=pl.BlockSpec((_TM // _NS, _COUT), lambda i: (i, 0)),
        out_shape=jax.ShapeDtypeStruct((nrows // _NS, _COUT), jnp.float32),
    )(gfeat, gx, w1f, w1x, b1, w2, b2, w3, b3)


def kernel(xyz, features, W1, b1, W2, b2, W3, b3):
    xr = xyz.transpose(0, 2, 1).reshape(_B * 3, 64, 128)
    idx8a, xyz16a, ds, qs = _fps_a(xr)
    idx8b, xyz16b = _fps_b(xr, ds, qs)
    idx8 = jnp.concatenate([idx8a, idx8b], axis=0)
    idx_fps = idx8[:, :_B].T.astype(jnp.int64)

    xyz_t = xyz.transpose(0, 2, 1).reshape(_B * 3 * _N)
    ft = features.transpose(0, 2, 1).reshape(_B * _N, _C)
    w1f, w1x = W1[3:], W1[:3]
    b1r, b2r, b3r = b1.reshape(1, -1), b2.reshape(1, -1), b3.reshape(1, -1)

    new_xyz_halves, idx_halves, nf_halves = [], [], []
    for x16 in (xyz16a, xyz16b):
        nxh = x16[:, :12].reshape(_SH, 3, _B).transpose(2, 0, 1)
        new_xyz_halves.append(nxh)
        q16 = jnp.pad(nxh.reshape(_B * _SH, 3), ((0, 0), (0, 13)))
        q16 = (q16.reshape(_B, _QW, _WPB, 16).transpose(0, 2, 1, 3)
               .reshape(_B * _SH, 16))
        idxfh, idxgh, gxfh = _ballq(xyz_t, q16)
        gfeath = _gather(ft, idxgh)
        gxh = gxfh.reshape(_B * _SH * _NS, 3)
        pooledh = _mlp(gfeath, gxh, w1f, w1x, b1r, W2, b2r, W3, b3r)
        idx_halves.append(idxfh.reshape(_B, _WPB, _QW, _NS)
                          .transpose(0, 2, 1, 3).reshape(_B, _SH, _NS))
        nf_halves.append(pooledh.reshape(_B, _WPB, _QW, _COUT)
                         .transpose(0, 3, 2, 1).reshape(_B, _COUT, _SH))

    new_xyz = jnp.concatenate(new_xyz_halves, axis=1)
    idx = jnp.concatenate(idx_halves, axis=1)
    new_features = jnp.concatenate(nf_halves, axis=2)
    return (new_xyz, idx_fps, new_features, idx)
